# Initial kernel scaffold; baseline (speedup 1.0000x reference)
import jax
import jax.numpy as jnp
from jax import lax
from jax.experimental import pallas as pl
from jax.experimental.pallas import tpu as pltpu
from jax.experimental.pallas import tpu_sc as plsc

_N0, _N1, _N2, _C = 10000, 20000, 10000, 256
_NC, _NS = 2, 16
_EB = 80
_ZR = 125


def _mm_kernel(x_ref, th_ref, o_ref):
    o_ref[0] = lax.dot_general(
        x_ref[...], th_ref[...], (((1,), (0,)), ((), ())),
        preferred_element_type=jnp.float32)


def _mm_sliced(x, th, s_slices, cs, bn=2000):
    n = x.shape[0]
    out = pl.pallas_call(
        _mm_kernel,
        grid=(s_slices, n // bn),
        in_specs=[
            pl.BlockSpec((bn, _C), lambda s, i: (i, 0)),
            pl.BlockSpec((_C, cs), lambda s, i: (0, s)),
        ],
        out_specs=pl.BlockSpec((1, bn, cs), lambda s, i: (s, i, 0)),
        out_shape=jax.ShapeDtypeStruct((s_slices, n, cs), jnp.float32),
    )(x, th)
    return out.reshape(s_slices * n, cs)


def _pad_edges(r, c, v, m):
    pad = (-r.shape[0]) % m
    if pad:
        r = jnp.concatenate([r, jnp.zeros((pad,), r.dtype)])
        c = jnp.concatenate([c, jnp.zeros((pad,), c.dtype)])
        v = jnp.concatenate([v, jnp.zeros((pad,), v.dtype)])
    return r, c, v


def _level(n_out, cs, s_slices, ops):
    tables = [o[0] for o in ops]
    n_ts = [o[1] for o in ops]
    edge_args = []
    nnz_ps = []
    for (_, _, r, c, v) in ops:
        r, c, v = _pad_edges(r, c, v, _EB * _NS)
        edge_args += [r, c, v]
        nnz_ps.append(r.shape[0])

    rows_pt = n_out // _NS
    nzch = rows_pt // _ZR
    half = s_slices // _NC
    mesh = plsc.VectorSubcoreMesh(core_axis_name="c", subcore_axis_name="s")

    def body(*refs):
        it = iter(refs)
        tab_refs = [next(it) for _ in ops]
        e_refs = [(next(it), next(it), next(it)) for _ in ops]
        out_ref = next(it)
        idx_v = next(it)
        r_v = next(it)
        v_v = next(it)
        rows_v = next(it)
        zs_v = next(it)
        sg_v = next(it)
        acc = next(it)
        sem = next(it)

        cid = lax.axis_index("c")
        sid = lax.axis_index("s")
        iota16 = lax.broadcasted_iota(jnp.int32, (16,), 0)

        def _zf(i, _):
            for k in range(cs // 16):
                zs_v[i, pl.ds(k * 16, 16)] = jnp.zeros((16,), jnp.float32)
            return 0
        lax.fori_loop(0, _ZR, _zf, 0)

        def run_slice(s):
            col0 = s * cs

            def _zc(chunk, _):
                row0 = sid * rows_pt + chunk * _ZR
                pltpu.sync_copy(zs_v, acc.at[pl.ds(row0, _ZR)])
                return 0
            lax.fori_loop(0, nzch, _zc, 0)
            plsc.subcore_barrier()

            for oi in range(len(ops)):
                tab = tab_refs[oi]
                r_hbm, c_hbm, v_hbm = e_refs[oi]
                nbt = nnz_ps[oi] // _EB // _NS
                off = s * n_ts[oi]

                def _batch(k, _, tab=tab, r_hbm=r_hbm, c_hbm=c_hbm,
                           v_hbm=v_hbm, nbt=nbt, off=off):
                    base = (sid * nbt + k) * _EB
                    pltpu.sync_copy(c_hbm.at[pl.ds(base, _EB)], idx_v)
                    pltpu.sync_copy(v_hbm.at[pl.ds(base, _EB)], v_v)
                    pltpu.sync_copy(r_hbm.at[pl.ds(base, _EB)], r_v)
                    for g in range(_EB // 16):
                        sl = pl.ds(g * 16, 16)
                        idx_v[sl] = idx_v[sl] + off
                    pltpu.async_copy(tab.at[idx_v], rows_v, sem).wait()

                    def _grp(g, _):
                        vv = v_v[pl.ds(g * 16, 16)]
                        rid = iota16 + g * 16
                        for ch in range(cs):
                            colv = jnp.full((16,), ch, jnp.int32)
                            xg = plsc.load_gather(rows_v, [rid, colv])
                            plsc.store_scatter(rows_v, [rid, colv], xg * vv)
                        return 0
                    lax.fori_loop(0, _EB // 16, _grp, 0)

                    pltpu.sync_copy(rows_v, acc.at[r_v], add=True)
                    return 0
                lax.fori_loop(0, nbt, _batch, 0)
            plsc.subcore_barrier()

            def _sg(chunk, _):
                row0 = sid * rows_pt + chunk * _ZR
                pltpu.sync_copy(acc.at[pl.ds(row0, _ZR)], sg_v)

                def _row(i, _):
                    for k in range(cs // 16):
                        sl = pl.ds(k * 16, 16)
                        x = sg_v[i, sl]
                        sg_v[i, sl] = 1.0 / (1.0 + jnp.exp(-x))
                    return 0
                lax.fori_loop(0, _ZR, _row, 0)
                pltpu.sync_copy(
                    sg_v, out_ref.at[pl.ds(row0, _ZR), pl.ds(col0, cs)])
                return 0
            lax.fori_loop(0, nzch, _sg, 0)
            plsc.subcore_barrier()

        for c in range(_NC):
            @pl.when(cid == c)
            def _(c=c):
                for s in range(c * half, (c + 1) * half):
                    run_slice(s)

    kern = pl.kernel(
        body,
        out_type=jax.ShapeDtypeStruct((n_out, _C), jnp.float32),
        mesh=mesh,
        scratch_types=[
            pltpu.VMEM((_EB,), jnp.int32),
            pltpu.VMEM((_EB,), jnp.int32),
            pltpu.VMEM((_EB,), jnp.float32),
            pltpu.VMEM((_EB, cs), jnp.float32),
            pltpu.VMEM((_ZR, cs), jnp.float32),
            pltpu.VMEM((_ZR, cs), jnp.float32),
            pltpu.VMEM_SHARED((n_out, cs), jnp.float32),
            pltpu.SemaphoreType.DMA,
        ],
    )
    return kern(*tables, *edge_args)


def kernel(x_0, x_1, x_2, th00, th10, th01, th11, th21, th12, th22,
           a0_r, a0_c, a0_v, b1_r, b1_c, b1_v, b1t_r, b1t_c, b1t_v,
           a1_r, a1_c, a1_v, b2_r, b2_c, b2_v, b2t_r, b2t_c, b2t_v,
           a2_r, a2_c, a2_v):
    t00 = _mm_sliced(x_0, th00, 2, 128)
    t10 = _mm_sliced(x_1, th10, 2, 128)
    t01 = _mm_sliced(x_0, th01, 4, 64)
    t11 = _mm_sliced(x_1, th11, 4, 64)
    t21 = _mm_sliced(x_2, th21, 4, 64)
    t12 = _mm_sliced(x_1, th12, 2, 128)
    t22 = _mm_sliced(x_2, th22, 2, 128)

    h0 = _level(_N0, 128, 2, [
        (t00, _N0, a0_r, a0_c, a0_v),
        (t10, _N1, b1_r, b1_c, b1_v),
    ])
    h1 = _level(_N1, 64, 4, [
        (t01, _N0, b1t_r, b1t_c, b1t_v),
        (t11, _N1, a1_r, a1_c, a1_v),
        (t21, _N2, b2_r, b2_c, b2_v),
    ])
    h2 = _level(_N2, 128, 2, [
        (t12, _N1, b2t_r, b2t_c, b2t_v),
        (t22, _N2, a2_r, a2_c, a2_v),
    ])
    return h0, h1, h2


# speedup vs baseline: 1.4825x; 1.4825x over previous
import jax
import jax.numpy as jnp
from jax import lax
from jax.experimental import pallas as pl
from jax.experimental.pallas import tpu as pltpu
from jax.experimental.pallas import tpu_sc as plsc

_N0, _N1, _N2, _C = 10000, 20000, 10000, 256
_NC, _NS = 2, 16
_EB = 80
_ZR = 40


def _mm_kernel(x_ref, th_ref, o_ref):
    o_ref[0] = lax.dot_general(
        x_ref[...], th_ref[0], (((1,), (0,)), ((), ())),
        preferred_element_type=jnp.float32)


def _mm_sliced(x, th, s_slices, cs, bn=2000):
    n = x.shape[0]
    th_s = th.reshape(_C, s_slices, cs).transpose(1, 0, 2)
    out = pl.pallas_call(
        _mm_kernel,
        grid=(s_slices, n // bn),
        in_specs=[
            pl.BlockSpec((bn, _C), lambda s, i: (i, 0)),
            pl.BlockSpec((1, _C, cs), lambda s, i: (s, 0, 0)),
        ],
        out_specs=pl.BlockSpec((1, bn, cs), lambda s, i: (s, i, 0)),
        out_shape=jax.ShapeDtypeStruct((s_slices, n, cs), jnp.float32),
    )(x, th_s)
    return out.reshape(s_slices * n, cs)


def _pad_edges(r, c, v, m):
    pad = (-r.shape[0]) % m
    if pad:
        r = jnp.concatenate([r, jnp.zeros((pad,), r.dtype)])
        c = jnp.concatenate([c, jnp.zeros((pad,), c.dtype)])
        v = jnp.concatenate([v, jnp.zeros((pad,), v.dtype)])
    return r, c, v


def _level(n_out, n_pad, n_acc, ops):
    cs = 128
    tables = [o[0] for o in ops]
    n_ts = [o[1] for o in ops]
    edge_args = []
    nnz_ps = []
    for (_, _, r, c, v) in ops:
        r, c, v = _pad_edges(r, c, v, _EB * _NS)
        edge_args += [r, c, v]
        nnz_ps.append(r.shape[0])

    passes = n_pad // n_acc
    rows_pt = n_acc // _NS
    nzch = rows_pt // _ZR
    mesh = plsc.VectorSubcoreMesh(
        core_axis_name="c", subcore_axis_name="s",
        num_cores=_NC, num_subcores=_NS)

    def body(*refs):
        it = iter(refs)
        tab_refs = [next(it) for _ in ops]
        e_refs = [(next(it), next(it), next(it)) for _ in ops]
        out_ref = next(it)
        idx_v = next(it)
        r_v = next(it)
        v_v = next(it)
        rows_v = next(it)
        zs_v = next(it)
        sg_v = next(it)
        acc = next(it)
        sem = next(it)

        cid = lax.axis_index("c")
        sid = lax.axis_index("s")

        def _zf(i, _):
            for k in range(cs // 16):
                zs_v[i, pl.ds(k * 16, 16)] = jnp.zeros((16,), jnp.float32)
            return 0
        lax.fori_loop(0, _ZR, _zf, 0)

        def run_pass(s, p):
            lo = p * n_acc

            def _zc(chunk, _):
                row0 = sid * rows_pt + chunk * _ZR
                pltpu.sync_copy(zs_v, acc.at[pl.ds(row0, _ZR)])
                return 0
            lax.fori_loop(0, nzch, _zc, 0)
            plsc.subcore_barrier()

            for oi in range(len(ops)):
                tab = tab_refs[oi]
                r_hbm, c_hbm, v_hbm = e_refs[oi]
                nbt = nnz_ps[oi] // _EB // _NS
                off = s * n_ts[oi]

                def _batch(k, _, tab=tab, r_hbm=r_hbm, c_hbm=c_hbm,
                           v_hbm=v_hbm, nbt=nbt, off=off):
                    base = (sid * nbt + k) * _EB
                    pltpu.sync_copy(c_hbm.at[pl.ds(base, _EB)], idx_v)
                    pltpu.sync_copy(v_hbm.at[pl.ds(base, _EB)], v_v)
                    pltpu.sync_copy(r_hbm.at[pl.ds(base, _EB)], r_v)
                    for g in range(_EB // 16):
                        sl = pl.ds(g * 16, 16)
                        idx_v[sl] = idx_v[sl] + off
                        if passes > 1:
                            rr = r_v[sl] - lo
                            ok = (rr >= 0) & (rr < n_acc)
                            r_v[sl] = jnp.where(
                                ok, rr, jnp.full((16,), n_acc, jnp.int32))
                    pltpu.async_copy(tab.at[idx_v], rows_v, sem).wait()

                    def _grp(g, _):
                        vv = v_v[pl.ds(g * 16, 16)]
                        for b in range(16):
                            splat = vv.at[
                                jnp.full((16,), b, jnp.int32)].get(
                                    mode="promise_in_bounds")
                            row = g * 16 + b
                            for k2 in range(cs // 16):
                                sl = pl.ds(k2 * 16, 16)
                                rows_v[row, sl] = rows_v[row, sl] * splat
                        return 0
                    lax.fori_loop(0, _EB // 16, _grp, 0)

                    pltpu.sync_copy(rows_v, acc.at[r_v], add=True)
                    return 0
                lax.fori_loop(0, nbt, _batch, 0)
            plsc.subcore_barrier()

            def _sg(chunk, _):
                row0 = sid * rows_pt + chunk * _ZR
                pltpu.sync_copy(acc.at[pl.ds(row0, _ZR)], sg_v)

                def _row(i, _):
                    for k in range(cs // 16):
                        sl = pl.ds(k * 16, 16)
                        x = sg_v[i, sl]
                        sg_v[i, sl] = 1.0 / (1.0 + jnp.exp(-x))
                    return 0
                lax.fori_loop(0, _ZR, _row, 0)
                pltpu.sync_copy(
                    sg_v, out_ref.at[pl.ds(s * n_pad + lo + row0, _ZR)])
                return 0
            lax.fori_loop(0, nzch, _sg, 0)
            plsc.subcore_barrier()

        for c in range(_NC):
            @pl.when(cid == c)
            def _(c=c):
                for p in range(passes):
                    run_pass(c, p)

    kern = pl.kernel(
        body,
        out_type=jax.ShapeDtypeStruct((2 * n_pad, cs), jnp.float32),
        mesh=mesh,
        scratch_types=[
            pltpu.VMEM((_EB,), jnp.int32),
            pltpu.VMEM((_EB,), jnp.int32),
            pltpu.VMEM((_EB,), jnp.float32),
            pltpu.VMEM((_EB, cs), jnp.float32),
            pltpu.VMEM((_ZR, cs), jnp.float32),
            pltpu.VMEM((_ZR, cs), jnp.float32),
            pltpu.VMEM_SHARED((n_acc + 8, cs), jnp.float32),
            pltpu.SemaphoreType.DMA,
        ],
    )
    out = kern(*tables, *edge_args)
    out = out.reshape(2, n_pad, cs).transpose(1, 0, 2)
    return out.reshape(n_pad, 2 * cs)[:n_out]


def kernel(x_0, x_1, x_2, th00, th10, th01, th11, th21, th12, th22,
           a0_r, a0_c, a0_v, b1_r, b1_c, b1_v, b1t_r, b1t_c, b1t_v,
           a1_r, a1_c, a1_v, b2_r, b2_c, b2_v, b2t_r, b2t_c, b2t_v,
           a2_r, a2_c, a2_v):
    t00 = _mm_sliced(x_0, th00, 2, 128)
    t10 = _mm_sliced(x_1, th10, 2, 128)
    t01 = _mm_sliced(x_0, th01, 2, 128)
    t11 = _mm_sliced(x_1, th11, 2, 128)
    t21 = _mm_sliced(x_2, th21, 2, 128)
    t12 = _mm_sliced(x_1, th12, 2, 128)
    t22 = _mm_sliced(x_2, th22, 2, 128)

    h0 = _level(_N0, 10240, 10240, [
        (t00, _N0, a0_r, a0_c, a0_v),
        (t10, _N1, b1_r, b1_c, b1_v),
    ])
    h1 = _level(_N1, 20480, 10240, [
        (t01, _N0, b1t_r, b1t_c, b1t_v),
        (t11, _N1, a1_r, a1_c, a1_v),
        (t21, _N2, b2_r, b2_c, b2_v),
    ])
    h2 = _level(_N2, 10240, 10240, [
        (t12, _N1, b2t_r, b2t_c, b2t_v),
        (t22, _N2, a2_r, a2_c, a2_v),
    ])
    return h0, h1, h2


# speedup vs baseline: 2.0368x; 1.3739x over previous
import jax
import jax.numpy as jnp
from jax import lax
from jax.experimental import pallas as pl
from jax.experimental.pallas import tpu as pltpu
from jax.experimental.pallas import tpu_sc as plsc

_N0, _N1, _N2, _C = 10000, 20000, 10000, 256
_NC, _NS = 2, 16
_EB = 80
_CB = 4
_CE = _EB * _CB
_ZR = 40


def _mm_kernel(x_ref, th_ref, o_ref):
    o_ref[0] = lax.dot_general(
        x_ref[...], th_ref[0], (((1,), (0,)), ((), ())),
        preferred_element_type=jnp.float32)


def _mm_sliced(x, th, bn=2000):
    n = x.shape[0]
    th_s = th.reshape(_C, 2, 128).transpose(1, 0, 2)
    out = pl.pallas_call(
        _mm_kernel,
        grid=(2, n // bn),
        in_specs=[
            pl.BlockSpec((bn, _C), lambda s, i: (i, 0)),
            pl.BlockSpec((1, _C, 128), lambda s, i: (s, 0, 0)),
        ],
        out_specs=pl.BlockSpec((1, bn, 128), lambda s, i: (s, i, 0)),
        out_shape=jax.ShapeDtypeStruct((2, n, 128), jnp.float32),
    )(x, th_s)
    return out.reshape(2 * n, 128)


def _pad_edges(r, c, v, m):
    pad = (-r.shape[0]) % m
    if pad:
        r = jnp.concatenate([r, jnp.zeros((pad,), r.dtype)])
        c = jnp.concatenate([c, jnp.zeros((pad,), c.dtype)])
        v = jnp.concatenate([v, jnp.zeros((pad,), v.dtype)])
    return r, c, v


def _level(n_out, n_pad, n_acc, ops):
    cs = 128
    tables = [o[0] for o in ops]
    n_ts = [o[1] for o in ops]
    edge_args = []
    nnz_ps = []
    for (_, _, r, c, v) in ops:
        r, c, v = _pad_edges(r, c, v, _CE * _NS)
        edge_args += [r, c, v]
        nnz_ps.append(r.shape[0])

    passes = n_pad // n_acc
    rows_pt = n_acc // _NS
    nzch = rows_pt // _ZR
    mesh = plsc.VectorSubcoreMesh(
        core_axis_name="c", subcore_axis_name="s",
        num_cores=_NC, num_subcores=_NS)

    def body(*refs):
        it = iter(refs)
        tab_refs = [next(it) for _ in ops]
        e_refs = [(next(it), next(it), next(it)) for _ in ops]
        out_ref = next(it)
        cch = next(it)
        rch = next(it)
        vch = next(it)
        i_bufs = (next(it), next(it))
        rsc = next(it)
        row_bufs = (next(it), next(it))
        zs_v = next(it)
        sg_v = next(it)
        acc = next(it)
        esem = next(it)
        gsems = (next(it), next(it))

        cid = lax.axis_index("c")
        sid = lax.axis_index("s")
        col0 = pl.multiple_of(cid * cs, cs)
        vdump = jnp.full((16,), n_acc, jnp.int32)

        def _zf(i, _):
            for k in range(cs // 16):
                zs_v[i, pl.ds(k * 16, 16)] = jnp.zeros((16,), jnp.float32)
            return 0
        lax.fori_loop(0, _ZR, _zf, 0)

        def run_pass(p):
            lo = p * n_acc
            vlo = jnp.full((16,), lo, jnp.int32)

            def _zc(chunk, _):
                row0 = pl.multiple_of(sid * rows_pt + chunk * _ZR, 8)
                pltpu.sync_copy(zs_v, acc.at[pl.ds(row0, _ZR)])
                return 0
            lax.fori_loop(0, nzch, _zc, 0)
            plsc.subcore_barrier()

            for oi in range(len(ops)):
                tab = tab_refs[oi]
                r_hbm, c_hbm, v_hbm = e_refs[oi]
                nbt = nnz_ps[oi] // _EB // _NS
                voff = jnp.full((16,), cid * n_ts[oi], jnp.int32)

                def _prep_idx(kk, voff=voff):
                    ib = i_bufs[kk % 2]
                    for g in range(_EB // 16):
                        sl = pl.ds(g * 16, 16)
                        ib[sl] = cch[pl.ds(kk * _EB + g * 16, 16)] + voff
                    return ib

                def _chunk(ci, _, tab=tab, r_hbm=r_hbm, c_hbm=c_hbm,
                           v_hbm=v_hbm, nbt=nbt, _prep_idx=_prep_idx):
                    base = (sid * nbt + ci * _CB) * _EB
                    d1 = pltpu.async_copy(
                        c_hbm.at[pl.ds(base, _CE)], cch, esem)
                    d2 = pltpu.async_copy(
                        r_hbm.at[pl.ds(base, _CE)], rch, esem)
                    d3 = pltpu.async_copy(
                        v_hbm.at[pl.ds(base, _CE)], vch, esem)
                    d1.wait()
                    d2.wait()
                    d3.wait()

                    ib0 = _prep_idx(0)
                    descs = {0: pltpu.async_copy(
                        tab.at[ib0], row_bufs[0], gsems[0])}
                    for kk in range(_CB):
                        rows = row_bufs[kk % 2]
                        if kk + 1 < _CB:
                            ibn = _prep_idx(kk + 1)
                            descs[kk + 1] = pltpu.async_copy(
                                tab.at[ibn], row_bufs[(kk + 1) % 2],
                                gsems[(kk + 1) % 2])
                        descs[kk].wait()
                        for g in range(_EB // 16):
                            src = rch[pl.ds(kk * _EB + g * 16, 16)]
                            if passes > 1:
                                rr = src - vlo
                                ok = (rr >= 0) & (rr < n_acc)
                                rsc[pl.ds(g * 16, 16)] = jnp.where(
                                    ok, rr, vdump)
                            else:
                                rsc[pl.ds(g * 16, 16)] = src

                        def _grp(g, _, rows=rows, kk=kk):
                            vv = vch[pl.ds(kk * _EB + g * 16, 16)]
                            for bb in range(16):
                                splat = vv.at[
                                    jnp.full((16,), bb, jnp.int32)].get(
                                        mode="promise_in_bounds")
                                row = g * 16 + bb
                                for k2 in range(cs // 16):
                                    sl = pl.ds(k2 * 16, 16)
                                    rows[row, sl] = rows[row, sl] * splat
                            return 0
                        lax.fori_loop(0, _EB // 16, _grp, 0)

                        pltpu.sync_copy(rows, acc.at[rsc], add=True)
                    return 0
                lax.fori_loop(0, nbt // _CB, _chunk, 0)
            plsc.subcore_barrier()

            def _sg(chunk, _):
                row0 = pl.multiple_of(sid * rows_pt + chunk * _ZR, 8)
                pltpu.sync_copy(acc.at[pl.ds(row0, _ZR)], sg_v)

                def _row(i, _):
                    for k in range(cs // 16):
                        sl = pl.ds(k * 16, 16)
                        x = sg_v[i, sl]
                        sg_v[i, sl] = 1.0 / (1.0 + jnp.exp(-x))
                    return 0
                lax.fori_loop(0, _ZR, _row, 0)
                rowg = pl.multiple_of(lo + row0, 8)
                pltpu.sync_copy(
                    sg_v, out_ref.at[pl.ds(rowg, _ZR), pl.ds(col0, cs)])
                return 0
            lax.fori_loop(0, nzch, _sg, 0)
            plsc.subcore_barrier()
            return 0

        if passes == 1:
            run_pass(0)
        else:
            lax.fori_loop(0, passes, lambda p, _: run_pass(p), 0)

    kern = pl.kernel(
        body,
        out_type=jax.ShapeDtypeStruct((n_pad, _C), jnp.float32),
        mesh=mesh,
        scratch_types=[
            pltpu.VMEM((_CE,), jnp.int32),
            pltpu.VMEM((_CE,), jnp.int32),
            pltpu.VMEM((_CE,), jnp.float32),
            pltpu.VMEM((_EB,), jnp.int32),
            pltpu.VMEM((_EB,), jnp.int32),
            pltpu.VMEM((_EB,), jnp.int32),
            pltpu.VMEM((_EB, cs), jnp.float32),
            pltpu.VMEM((_EB, cs), jnp.float32),
            pltpu.VMEM((_ZR, cs), jnp.float32),
            pltpu.VMEM((_ZR, cs), jnp.float32),
            pltpu.VMEM_SHARED((n_acc + 8, cs), jnp.float32),
            pltpu.SemaphoreType.DMA,
            pltpu.SemaphoreType.DMA,
            pltpu.SemaphoreType.DMA,
        ],
    )
    out = kern(*tables, *edge_args)
    return out[:n_out]


def kernel(x_0, x_1, x_2, th00, th10, th01, th11, th21, th12, th22,
           a0_r, a0_c, a0_v, b1_r, b1_c, b1_v, b1t_r, b1t_c, b1t_v,
           a1_r, a1_c, a1_v, b2_r, b2_c, b2_v, b2t_r, b2t_c, b2t_v,
           a2_r, a2_c, a2_v):
    t00 = _mm_sliced(x_0, th00)
    t10 = _mm_sliced(x_1, th10)
    t01 = _mm_sliced(x_0, th01)
    t11 = _mm_sliced(x_1, th11)
    t21 = _mm_sliced(x_2, th21)
    t12 = _mm_sliced(x_1, th12)
    t22 = _mm_sliced(x_2, th22)

    h0 = _level(_N0, 10240, 10240, [
        (t00, _N0, a0_r, a0_c, a0_v),
        (t10, _N1, b1_r, b1_c, b1_v),
    ])
    h1 = _level(_N1, 20480, 10240, [
        (t01, _N0, b1t_r, b1t_c, b1t_v),
        (t11, _N1, a1_r, a1_c, a1_v),
        (t21, _N2, b2_r, b2_c, b2_v),
    ])
    h2 = _level(_N2, 10240, 10240, [
        (t12, _N1, b2t_r, b2t_c, b2t_v),
        (t22, _N2, a2_r, a2_c, a2_v),
    ])
    return h0, h1, h2
